# Initial kernel scaffold; baseline (speedup 1.0000x reference)
import jax
import jax.numpy as jnp
from jax.experimental import pallas as pl


def kernel(x, noise):
    raise NotImplementedError("write your pallas kernel here")


# speedup vs baseline: 2.0498x; 2.0498x over previous
import functools

import jax
import jax.numpy as jnp
from jax.experimental import pallas as pl
from jax.experimental.pallas import tpu as pltpu

_MASK_RATIO = 0.75
_WINDOW = (16, 16, 16)


def _keep_kernel(nrow_ref, ncol_ref, keep_ref, *, num_windows, len_keep):
    n = nrow_ref[0]
    nc = ncol_ref[0]
    wp = jax.lax.broadcasted_iota(jnp.int32, (num_windows, num_windows), 0)
    wo = jax.lax.broadcasted_iota(jnp.int32, (num_windows, num_windows), 1)
    m = (nc < n) | ((nc == n) & (wp < wo))
    rank = jnp.sum(m.astype(jnp.int32), axis=0, keepdims=True)
    keep_ref[0] = (rank < len_keep).astype(jnp.float32)


def _mask_kernel(keep_ref, x_ref, xm_ref, mask_ref, *, nww, nwd, ww, DC):
    b = pl.program_id(0)
    i = pl.program_id(1)
    j = pl.program_id(2)
    base = i * (nww * nwd) + j * nwd
    kwin = jax.lax.broadcasted_iota(jnp.int32, (ww, DC), 1) // (DC // nwd)
    vis = jnp.zeros((ww, DC), jnp.float32)
    for k in range(nwd):
        kv = keep_ref[b, base + k]
        vis = jnp.where(kwin == k, kv, vis)
    visb = vis[None, None]
    xm_ref[...] = x_ref[...] * visb
    mask_ref[...] = jnp.broadcast_to(1.0 - visb, mask_ref.shape)


def kernel(x, noise):
    B, H, W, D, C = x.shape
    wh, ww, wd = _WINDOW
    assert H % wh == 0 and W % ww == 0 and D % wd == 0
    nwh, nww, nwd = H // wh, W // ww, D // wd
    num_windows = nwh * nww * nwd
    len_keep = int(num_windows * (1 - _MASK_RATIO))

    nrow = noise.reshape(B, 1, num_windows)
    ncol = noise.reshape(B, num_windows, 1)
    keep = pl.pallas_call(
        functools.partial(
            _keep_kernel, num_windows=num_windows, len_keep=len_keep
        ),
        grid=(B,),
        in_specs=[
            pl.BlockSpec((1, 1, num_windows), lambda b: (b, 0, 0)),
            pl.BlockSpec((1, num_windows, 1), lambda b: (b, 0, 0)),
        ],
        out_specs=pl.BlockSpec((1, 1, num_windows), lambda b: (b, 0, 0)),
        out_shape=jax.ShapeDtypeStruct((B, 1, num_windows), jnp.float32),
    )(nrow, ncol)
    keep = keep.reshape(B, num_windows)

    DC = D * C
    x4 = x.reshape(B, H, W, DC)
    x_masked, mask = pl.pallas_call(
        functools.partial(_mask_kernel, nww=nww, nwd=nwd, ww=ww, DC=DC),
        grid=(B, nwh, nww),
        in_specs=[
            pl.BlockSpec(memory_space=pltpu.SMEM),
            pl.BlockSpec((1, wh, ww, DC), lambda b, i, j: (b, i, j, 0)),
        ],
        out_specs=[
            pl.BlockSpec((1, wh, ww, DC), lambda b, i, j: (b, i, j, 0)),
            pl.BlockSpec((1, wh, ww, DC), lambda b, i, j: (b, i, j, 0)),
        ],
        out_shape=[
            jax.ShapeDtypeStruct((B, H, W, DC), x.dtype),
            jax.ShapeDtypeStruct((B, H, W, DC), x.dtype),
        ],
        compiler_params=pltpu.CompilerParams(
            dimension_semantics=("parallel", "parallel", "parallel"),
        ),
    )(keep, x4)
    shape5 = (B, H, W, D, C)
    return (x_masked.reshape(shape5), mask.reshape(shape5))
